# Initial kernel scaffold; baseline (speedup 1.0000x reference)
import jax
import jax.numpy as jnp
from jax.experimental import pallas as pl


def kernel(x, edge_index, pos, batch, W1a, b1a, W1b, b1b, g1, be1, W2a, b2a, W2b, b2b, g2, be2, W3a, b3a, W3b, b3b, g3, be3, W4a, b4a, W4b, b4b, g4, be4, W5a, b5a, W5b, b5b, g5, be5, W6a, b6a, W6b, b6b, g6, be6, W7a, b7a, W7b, b7b, g7, be7, Wfc):
    raise NotImplementedError("write your pallas kernel here")


# speedup vs baseline: 1.1949x; 1.1949x over previous
import functools
import jax
import jax.numpy as jnp
from jax import lax
from jax.experimental import pallas as pl
from jax.experimental.pallas import tpu as pltpu
from jax.experimental.pallas import tpu_sc as plsc

N_NODES = 100000
N_CLUST = 225


NTILE = 32
NPT = N_NODES // NTILE
SLACK = 4096
CHB = 4000
CHS = 512
BE = 8192
TOT2 = BE * 212


def _mesh():
    return plsc.VectorSubcoreMesh(core_axis_name="c", subcore_axis_name="s")


def _wid():
    return lax.axis_index("s") * 2 + lax.axis_index("c")


def _m8(x):
    return pl.multiple_of(x, 8)


def _lanesum(tmp, v):
    lanes = lax.iota(jnp.int32, 16)
    a = v
    for sh in (8, 4, 2, 1):
        tmp[...] = a
        a = a + plsc.load_gather(tmp, [(lanes + sh) & 15])
    return a[0]


def _prefix(cvm, w):
    def body(i, s):
        c = cvm[pl.ds(i * 16, 16)][0]
        ca = (c + 7) & (-8)
        return s + jnp.where(i < w, ca + SLACK, 0)
    return lax.fori_loop(0, NTILE, body, jnp.int32(0))


def _sc_hist(dst):
    E = dst.shape[0]
    assert E % CHB == 0

    @functools.partial(
        pl.kernel, mesh=_mesh(),
        compiler_params=pltpu.CompilerParams(needs_layout_passes=False),
        out_type=jax.ShapeDtypeStruct((NTILE * 16,), jnp.int32),
        scratch_types=[pltpu.VMEM((CHB,), jnp.int32),
                       pltpu.VMEM((16,), jnp.int32)])
    def k(dst_hbm, out_hbm, dbuf, cvec):
        w = _wid()
        lo = w * NPT
        hi = lo + NPT

        def chunk(ci, tot):
            pltpu.sync_copy(dst_hbm.at[pl.ds(_m8(ci * CHB), CHB)], dbuf)

            def vec(j, cv):
                d = dbuf[pl.ds(j * 16, 16)]
                m = (d >= lo) & (d < hi)
                return cv + jnp.where(m, 1, 0)

            cv = lax.fori_loop(0, CHB // 16, vec, jnp.zeros((16,), jnp.int32))
            return tot + _lanesum(cvec, cv)

        tot = lax.fori_loop(0, E // CHB, chunk, jnp.int32(0))
        cvec[...] = jnp.full((16,), tot, jnp.int32)
        pltpu.sync_copy(cvec, out_hbm.at[pl.ds(_m8(w * 16), 16)])

    return k(dst)


def _sc_part(src, dst, counts):
    E = src.shape[0]
    assert E % CHB == 0

    @functools.partial(
        pl.kernel, mesh=_mesh(),
        compiler_params=pltpu.CompilerParams(needs_layout_passes=False),
        out_type=(jax.ShapeDtypeStruct((TOT2,), jnp.int32),
                  jax.ShapeDtypeStruct((TOT2,), jnp.int32)),
        scratch_types=[pltpu.VMEM((NTILE * 16,), jnp.int32),
                       pltpu.VMEM((CHB,), jnp.int32),
                       pltpu.VMEM((CHB,), jnp.int32),
                       pltpu.VMEM((4112,), jnp.int32),
                       pltpu.VMEM((4112,), jnp.int32),
                       pltpu.VMEM((16,), jnp.int32)])
    def k(src_hbm, dst_hbm, cnt_hbm, ssrc_hbm, sdst_hbm,
          cvm, sbuf, dbuf, obs, obd, tmp16):
        w = _wid()
        lo = w * NPT
        hi = lo + NPT
        pltpu.sync_copy(cnt_hbm, cvm)
        start = _prefix(cvm, w)

        def chunk(ci, carry):
            pltpu.sync_copy(src_hbm.at[pl.ds(_m8(ci * CHB), CHB)], sbuf)
            pltpu.sync_copy(dst_hbm.at[pl.ds(_m8(ci * CHB), CHB)], dbuf)

            def vec(j, c2):
                gout2, fill2 = c2
                d = dbuf[pl.ds(j * 16, 16)]
                s = sbuf[pl.ds(j * 16, 16)]
                m = (d >= lo) & (d < hi)
                plsc.store_compressed(obd.at[pl.ds(fill2, 16)], d, mask=m)
                plsc.store_compressed(obs.at[pl.ds(fill2, 16)], s, mask=m)
                fill3 = fill2 + _lanesum(tmp16, jnp.where(m, 1, 0))
                do_flush = fill3 >= 4096

                @pl.when(do_flush)
                def _():
                    pltpu.sync_copy(obd.at[pl.ds(0, 4096)],
                                    sdst_hbm.at[pl.ds(_m8(gout2), 4096)])
                    pltpu.sync_copy(obs.at[pl.ds(0, 4096)],
                                    ssrc_hbm.at[pl.ds(_m8(gout2), 4096)])
                    obd[pl.ds(0, 16)] = obd[pl.ds(4096, 16)]
                    obs[pl.ds(0, 16)] = obs[pl.ds(4096, 16)]

                gout3 = gout2 + jnp.where(do_flush, 4096, 0)
                fill4 = fill3 - jnp.where(do_flush, 4096, 0)
                return (gout3, fill4)

            return lax.fori_loop(0, CHB // 16, vec, carry)

        gout, fill = lax.fori_loop(0, E // CHB, chunk,
                                   (start, jnp.int32(0)))
        pltpu.sync_copy(obd.at[pl.ds(0, 4096)],
                        sdst_hbm.at[pl.ds(_m8(gout), 4096)])
        pltpu.sync_copy(obs.at[pl.ds(0, 4096)],
                        ssrc_hbm.at[pl.ds(_m8(gout), 4096)])

    return k(src, dst, counts)


def _sc_scatter(H, sdstg, counts, cout):
    Hf = H.reshape(-1)

    @functools.partial(
        pl.kernel, mesh=_mesh(),
        compiler_params=pltpu.CompilerParams(needs_layout_passes=False),
        out_type=jax.ShapeDtypeStruct((N_NODES * cout,), jnp.float32),
        scratch_types=[pltpu.VMEM((NTILE * 16,), jnp.int32),
                       pltpu.VMEM((CHS * cout,), jnp.float32),
                       pltpu.VMEM((CHS + 16,), jnp.int32),
                       pltpu.VMEM((NPT * cout,), jnp.float32)])
    def k(h_hbm, sd_hbm, cnt_hbm, out_hbm, cvm, hbuf, dbuf, acc):
        w = _wid()
        lo = w * NPT
        pltpu.sync_copy(cnt_hbm, cvm)
        start = _prefix(cvm, w)
        cnt = cvm[pl.ds(w * 16, 16)][0]

        ninf = jnp.full((16,), -jnp.inf, jnp.float32)

        def initv(i, _):
            acc[pl.ds(i * 16, 16)] = ninf
            return 0

        lax.fori_loop(0, NPT * cout // 16, initv, 0)

        nch = lax.shift_right_arithmetic(cnt + CHS - 1, 9)

        def chunk(ci, _):
            r0 = start + ci * CHS
            pltpu.sync_copy(h_hbm.at[pl.ds(_m8(r0 * cout), CHS * cout)], hbuf)
            pltpu.sync_copy(sd_hbm.at[pl.ds(_m8(r0), CHS)],
                            dbuf.at[pl.ds(0, CHS)])
            rem = cnt - ci * CHS

            def edge(e, _2):
                d = dbuf[pl.ds(e, 16)][0]
                li = jnp.clip(d - lo, 0, NPT - 1)
                base = li * cout
                hb = e * cout
                ok = e < rem
                for kk in range(cout // 16):
                    hv = hbuf[pl.ds(hb + kk * 16, 16)]
                    hv = jnp.where(ok, hv, ninf)
                    av = acc[pl.ds(base + kk * 16, 16)]
                    acc[pl.ds(base + kk * 16, 16)] = jnp.maximum(av, hv)
                return 0

            lax.fori_loop(0, CHS, edge, 0)
            return 0

        lax.fori_loop(0, nch, chunk, 0)
        pltpu.sync_copy(acc, out_hbm.at[pl.ds(_m8(lo * cout), NPT * cout)])

    return k(Hf, sdstg, counts)


def _sc_gather(y, ssrc, sdstg, counts):
    cin = y.shape[1]
    assert cin >= 16
    CHG = 128
    yin = jnp.zeros((y.shape[0], 128), jnp.float32).at[:, :cin].set(y)
    xshape = (CHG, 128)

    @functools.partial(
        pl.kernel, mesh=_mesh(),
        compiler_params=pltpu.CompilerParams(needs_layout_passes=False),
        out_type=jax.ShapeDtypeStruct((TOT2 * 2 * cin,), jnp.float32),
        scratch_types=[pltpu.VMEM((NTILE * 16,), jnp.int32),
                       pltpu.VMEM((CHG,), jnp.int32),
                       pltpu.VMEM((CHG,), jnp.int32),
                       pltpu.VMEM(xshape, jnp.float32),
                       pltpu.VMEM(xshape, jnp.float32),
                       pltpu.VMEM((CHG * 2 * cin,), jnp.float32),
                       pltpu.SemaphoreType.DMA,
                       pltpu.SemaphoreType.DMA])
    def k(y_hbm, src_hbm, dst_hbm, cnt_hbm, cat_hbm,
          cvm, ibuf, jbuf, xib, xjb, cbuf, sem1, sem2):
        w = _wid()
        pltpu.sync_copy(cnt_hbm, cvm)
        start = _prefix(cvm, w)
        cnt = cvm[pl.ds(w * 16, 16)][0]
        nch = lax.shift_right_arithmetic(cnt + CHG - 1, 7)
        lanes = lax.iota(jnp.int32, 16)

        def chunk(ci, _):
            r0 = start + ci * CHG
            pltpu.sync_copy(dst_hbm.at[pl.ds(_m8(r0), CHG)], ibuf)
            pltpu.sync_copy(src_hbm.at[pl.ds(_m8(r0), CHG)], jbuf)
            rem = cnt - ci * CHG

            def sanitize(j, _2):
                pos = j * 16 + lanes
                ok = pos < rem
                iv = ibuf[pl.ds(j * 16, 16)]
                jv = jbuf[pl.ds(j * 16, 16)]
                ibuf[pl.ds(j * 16, 16)] = jnp.where(ok, iv, 0)
                jbuf[pl.ds(j * 16, 16)] = jnp.where(ok, jv, 0)
                return 0

            lax.fori_loop(0, CHG // 16, sanitize, 0)
            cp1 = pltpu.async_copy(y_hbm.at[ibuf], xib, sem1)
            cp2 = pltpu.async_copy(y_hbm.at[jbuf], xjb, sem2)
            cp1.wait()
            cp2.wait()

            def row(r, _3):
                for kk in range(cin // 16):
                    xi = xib[r, pl.ds(kk * 16, 16)]
                    xj = xjb[r, pl.ds(kk * 16, 16)]
                    cbuf[pl.ds(r * 2 * cin + kk * 16, 16)] = xi
                    cbuf[pl.ds(r * 2 * cin + cin + kk * 16, 16)] = xj - xi
                return 0

            lax.fori_loop(0, CHG, row, 0)

            pltpu.sync_copy(cbuf,
                            cat_hbm.at[pl.ds(_m8(r0 * 2 * cin),
                                             CHG * 2 * cin)])
            return 0

        lax.fori_loop(0, nch, chunk, 0)

    return k(yin, ssrc, sdstg, counts).reshape(TOT2, 2 * cin)


def _edge_mlp_kernel(cat_ref, wa_ref, ba_ref, wb_ref, bb_ref, out_ref):
    h = jnp.maximum(lax.dot(cat_ref[...], wa_ref[...]) + ba_ref[...], 0.0)
    out_ref[...] = lax.dot(h, wb_ref[...]) + bb_ref[...]


def _edge_mlp(CAT, Wa, ba, Wb, bb):
    Ep, k2 = CAT.shape
    cm = Wa.shape[1]
    n = Wb.shape[1]
    assert Ep % BE == 0
    return pl.pallas_call(
        _edge_mlp_kernel,
        grid=(Ep // BE,),
        in_specs=[pl.BlockSpec((BE, k2), lambda i: (i, 0)),
                  pl.BlockSpec((k2, cm), lambda i: (0, 0)),
                  pl.BlockSpec((1, cm), lambda i: (0, 0)),
                  pl.BlockSpec((cm, n), lambda i: (0, 0)),
                  pl.BlockSpec((1, n), lambda i: (0, 0))],
        out_specs=pl.BlockSpec((BE, n), lambda i: (i, 0)),
        out_shape=jax.ShapeDtypeStruct((Ep, n), jnp.float32),
    )(CAT, Wa, ba.reshape(1, -1), Wb, bb.reshape(1, -1))


def _edge_conv_fine(y, ssrc, sdstg, counts, Wa, ba, Wb, bb):
    cout = Wb.shape[1]
    cpad = max(16, cout)
    if cpad != cout:
        Wbp = jnp.zeros((Wb.shape[0], cpad), jnp.float32).at[:, :cout].set(Wb)
        bbp = jnp.zeros((cpad,), jnp.float32).at[:cout].set(bb)
    else:
        Wbp, bbp = Wb, bb
    if y.shape[1] >= 16:
        CAT = _sc_gather(y, ssrc, sdstg, counts)
    else:
        XI = jnp.take(y, sdstg, axis=0, mode="clip")
        XJ = jnp.take(y, ssrc, axis=0, mode="clip")
        CAT = jnp.concatenate([XI, XJ - XI], axis=1)
    H = _edge_mlp(CAT, Wa, ba, Wbp, bbp)
    acc = _sc_scatter(H, sdstg, counts, cpad).reshape(N_NODES, cpad)[:, :cout]
    return jnp.where(jnp.isfinite(acc), acc, 0.0)


def _bn(x, g, b):
    m = jnp.mean(x, axis=0)
    v = jnp.var(x, axis=0)
    return (x - m) * lax.rsqrt(v + 1e-5) * g + b


PD = 232
SP = 256


def _pair_kernel(cat_ref, mask_ref, wa_ref, ba_ref, wb_ref, bb_ref, out_ref):
    h = jnp.maximum(lax.dot(cat_ref[...], wa_ref[...]) + ba_ref[...], 0.0)
    h = lax.dot(h, wb_ref[...]) + bb_ref[...]
    h = jnp.where(mask_ref[...] > 0.5, h, -jnp.inf)
    for i in range(8):
        mx = jnp.max(h[i * SP:(i + 1) * SP], axis=0, keepdims=True)
        out_ref[i:i + 1, :] = jnp.where(jnp.isfinite(mx), mx, 0.0)


def _pair_conv(hin, maskf, Wa, ba, Wb, bb):
    cm = Wa.shape[1]
    n = Wb.shape[1]
    hpad = jnp.zeros((SP, 32), jnp.float32).at[:N_CLUST].set(hin)
    xi = jnp.broadcast_to(hpad[:PD, None, :], (PD, SP, 32))
    xj = jnp.broadcast_to(hpad[None, :, :], (PD, SP, 32))
    cat = jnp.concatenate([xi, xj - xi], axis=2).reshape(PD * SP, 64)
    out = pl.pallas_call(
        _pair_kernel,
        grid=(PD // 8,),
        in_specs=[pl.BlockSpec((8 * SP, 64), lambda i: (i, 0)),
                  pl.BlockSpec((8 * SP, 1), lambda i: (i, 0)),
                  pl.BlockSpec((64, cm), lambda i: (0, 0)),
                  pl.BlockSpec((1, cm), lambda i: (0, 0)),
                  pl.BlockSpec((cm, n), lambda i: (0, 0)),
                  pl.BlockSpec((1, n), lambda i: (0, 0))],
        out_specs=pl.BlockSpec((8, n), lambda i: (i, 0)),
        out_shape=jax.ShapeDtypeStruct((PD, n), jnp.float32),
    )(cat, maskf, Wa, ba.reshape(1, -1), Wb, bb.reshape(1, -1))
    return out[:N_CLUST]


def kernel(x, edge_index, pos, batch, W1a, b1a, W1b, b1b, g1, be1, W2a, b2a,
           W2b, b2b, g2, be2, W3a, b3a, W3b, b3b, g3, be3, W4a, b4a, W4b, b4b,
           g4, be4, W5a, b5a, W5b, b5b, g5, be5, W6a, b6a, W6b, b6b, g6, be6,
           W7a, b7a, W7b, b7b, g7, be7, Wfc):
    src = edge_index[0]
    dst = edge_index[1]
    elu = jax.nn.elu

    counts = _sc_hist(dst)
    ssrc, sdstg = _sc_part(src, dst, counts)

    h = _bn(elu(_edge_conv_fine(x, ssrc, sdstg, counts, W1a, b1a, W1b, b1b)), g1, be1)
    h = _bn(elu(_edge_conv_fine(h, ssrc, sdstg, counts, W2a, b2a, W2b, b2b)), g2, be2)
    sc = h
    h = _bn(elu(_edge_conv_fine(h, ssrc, sdstg, counts, W3a, b3a, W3b, b3b)), g3, be3)
    h = _bn(elu(_edge_conv_fine(h, ssrc, sdstg, counts, W4a, b4a, W4b, b4b)), g4, be4)
    h = h + sc
    h = _bn(elu(_edge_conv_fine(h, ssrc, sdstg, counts, W5a, b5a, W5b, b5b)), g5, be5)

    cx = jnp.floor(pos[:, 0] / 16.0).astype(jnp.int32)
    cy = jnp.floor(pos[:, 1] / 12.0).astype(jnp.int32)
    cluster = cx + 15 * cy
    hp = jax.ops.segment_max(h, cluster, num_segments=N_CLUST)
    hp = jnp.where(jnp.isfinite(hp), hp, 0.0)
    cnt = jax.ops.segment_sum(jnp.ones((N_NODES, 1), dtype=jnp.float32),
                              cluster, num_segments=N_CLUST)
    ppos = jax.ops.segment_sum(pos, cluster,
                               num_segments=N_CLUST) / jnp.maximum(cnt, 1.0)

    pair = cluster[dst] * N_CLUST + cluster[src]
    prescnt = jax.ops.segment_sum(jnp.ones((pair.shape[0],), jnp.float32),
                                  pair, num_segments=N_CLUST * N_CLUST)
    pres = (prescnt > 0.0).reshape(N_CLUST, N_CLUST)
    pres = pres & ~jnp.eye(N_CLUST, dtype=bool)
    maskf = jnp.zeros((PD, SP), jnp.float32).at[:N_CLUST, :N_CLUST].set(
        pres.astype(jnp.float32)).reshape(PD * SP, 1)

    sc = hp
    h = _bn(elu(_pair_conv(hp, maskf, W6a, b6a, W6b, b6b)), g6, be6)
    h = _bn(elu(_pair_conv(h, maskf, W7a, b7a, W7b, b7b)), g7, be7)
    h = h + sc

    gx = jnp.clip(jnp.floor(ppos[:, 0] / 60.0).astype(jnp.int32), 0, 3)
    gy = jnp.clip(jnp.floor(ppos[:, 1] / 45.0).astype(jnp.int32), 0, 3)
    cell = gx + 4 * gy
    occ = cnt[:, 0] > 0
    hm = jnp.where(occ[:, None], h, -jnp.inf)
    out = jax.ops.segment_max(hm, cell, num_segments=16)
    out = jnp.where(jnp.isfinite(out), out, 0.0)
    flat = out.reshape(1, 32 * 16)
    return flat @ Wfc


# speedup vs baseline: 2.1637x; 1.8108x over previous
import functools
import jax
import jax.numpy as jnp
from jax import lax
from jax.experimental import pallas as pl
from jax.experimental.pallas import tpu as pltpu
from jax.experimental.pallas import tpu_sc as plsc

N_NODES = 100000
N_CLUST = 225


NTILE = 32
NPT = N_NODES // NTILE
SLACK = 4096
CHB = 4000
CHS = 512
BE = 8192
TOT2 = BE * 212


def _mesh():
    return plsc.VectorSubcoreMesh(core_axis_name="c", subcore_axis_name="s")


def _wid():
    return lax.axis_index("s") * 2 + lax.axis_index("c")


def _m8(x):
    return pl.multiple_of(x, 8)


def _lanesum(tmp, v):
    lanes = lax.iota(jnp.int32, 16)
    a = v
    for sh in (8, 4, 2, 1):
        tmp[...] = a
        a = a + plsc.load_gather(tmp, [(lanes + sh) & 15])
    return a[0]


def _prefix(cvm, w):
    def body(i, s):
        c = cvm[pl.ds(i * 16, 16)][0]
        ca = (c + 7) & (-8)
        return s + jnp.where(i < w, ca + SLACK, 0)
    return lax.fori_loop(0, NTILE, body, jnp.int32(0))


def _sc_hist(dst):
    E = dst.shape[0]
    assert E % CHB == 0

    @functools.partial(
        pl.kernel, mesh=_mesh(),
        compiler_params=pltpu.CompilerParams(needs_layout_passes=False),
        out_type=jax.ShapeDtypeStruct((NTILE * 16,), jnp.int32),
        scratch_types=[pltpu.VMEM((CHB,), jnp.int32),
                       pltpu.VMEM((16,), jnp.int32)])
    def k(dst_hbm, out_hbm, dbuf, cvec):
        w = _wid()
        lo = w * NPT
        hi = lo + NPT

        def chunk(ci, tot):
            pltpu.sync_copy(dst_hbm.at[pl.ds(_m8(ci * CHB), CHB)], dbuf)

            def vec(j, cv):
                d = dbuf[pl.ds(j * 16, 16)]
                m = (d >= lo) & (d < hi)
                return cv + jnp.where(m, 1, 0)

            cv = lax.fori_loop(0, CHB // 16, vec, jnp.zeros((16,), jnp.int32))
            return tot + _lanesum(cvec, cv)

        tot = lax.fori_loop(0, E // CHB, chunk, jnp.int32(0))
        cvec[...] = jnp.full((16,), tot, jnp.int32)
        pltpu.sync_copy(cvec, out_hbm.at[pl.ds(_m8(w * 16), 16)])

    return k(dst)


def _sc_part(src, dst, counts):
    E = src.shape[0]
    assert E % CHB == 0

    @functools.partial(
        pl.kernel, mesh=_mesh(),
        compiler_params=pltpu.CompilerParams(needs_layout_passes=False),
        out_type=(jax.ShapeDtypeStruct((TOT2,), jnp.int32),
                  jax.ShapeDtypeStruct((TOT2,), jnp.int32)),
        scratch_types=[pltpu.VMEM((NTILE * 16,), jnp.int32),
                       pltpu.VMEM((CHB,), jnp.int32),
                       pltpu.VMEM((CHB,), jnp.int32),
                       pltpu.VMEM((4112,), jnp.int32),
                       pltpu.VMEM((4112,), jnp.int32),
                       pltpu.VMEM((16,), jnp.int32)])
    def k(src_hbm, dst_hbm, cnt_hbm, ssrc_hbm, sdst_hbm,
          cvm, sbuf, dbuf, obs, obd, tmp16):
        w = _wid()
        lo = w * NPT
        hi = lo + NPT
        pltpu.sync_copy(cnt_hbm, cvm)
        start = _prefix(cvm, w)

        def chunk(ci, carry):
            pltpu.sync_copy(src_hbm.at[pl.ds(_m8(ci * CHB), CHB)], sbuf)
            pltpu.sync_copy(dst_hbm.at[pl.ds(_m8(ci * CHB), CHB)], dbuf)

            def vec(j, c2):
                gout2, fill2 = c2
                d = dbuf[pl.ds(j * 16, 16)]
                s = sbuf[pl.ds(j * 16, 16)]
                m = (d >= lo) & (d < hi)
                plsc.store_compressed(obd.at[pl.ds(fill2, 16)], d, mask=m)
                plsc.store_compressed(obs.at[pl.ds(fill2, 16)], s, mask=m)
                fill3 = fill2 + _lanesum(tmp16, jnp.where(m, 1, 0))
                do_flush = fill3 >= 4096

                @pl.when(do_flush)
                def _():
                    pltpu.sync_copy(obd.at[pl.ds(0, 4096)],
                                    sdst_hbm.at[pl.ds(_m8(gout2), 4096)])
                    pltpu.sync_copy(obs.at[pl.ds(0, 4096)],
                                    ssrc_hbm.at[pl.ds(_m8(gout2), 4096)])
                    obd[pl.ds(0, 16)] = obd[pl.ds(4096, 16)]
                    obs[pl.ds(0, 16)] = obs[pl.ds(4096, 16)]

                gout3 = gout2 + jnp.where(do_flush, 4096, 0)
                fill4 = fill3 - jnp.where(do_flush, 4096, 0)
                return (gout3, fill4)

            return lax.fori_loop(0, CHB // 16, vec, carry)

        gout, fill = lax.fori_loop(0, E // CHB, chunk,
                                   (start, jnp.int32(0)))
        pltpu.sync_copy(obd.at[pl.ds(0, 4096)],
                        sdst_hbm.at[pl.ds(_m8(gout), 4096)])
        pltpu.sync_copy(obs.at[pl.ds(0, 4096)],
                        ssrc_hbm.at[pl.ds(_m8(gout), 4096)])

    return k(src, dst, counts)


def _sc_scatter(H, sdstg, counts, cout):
    Hf = H.reshape(-1)

    @functools.partial(
        pl.kernel, mesh=_mesh(),
        compiler_params=pltpu.CompilerParams(needs_layout_passes=False),
        out_type=jax.ShapeDtypeStruct((N_NODES * cout,), jnp.float32),
        scratch_types=[pltpu.VMEM((NTILE * 16,), jnp.int32),
                       pltpu.VMEM((CHS * cout,), jnp.float32),
                       pltpu.VMEM((CHS + 16,), jnp.int32),
                       pltpu.VMEM((NPT * cout,), jnp.float32)])
    def k(h_hbm, sd_hbm, cnt_hbm, out_hbm, cvm, hbuf, dbuf, acc):
        w = _wid()
        lo = w * NPT
        pltpu.sync_copy(cnt_hbm, cvm)
        start = _prefix(cvm, w)
        cnt = cvm[pl.ds(w * 16, 16)][0]

        ninf = jnp.full((16,), -jnp.inf, jnp.float32)

        def initv(i, _):
            acc[pl.ds(i * 16, 16)] = ninf
            return 0

        lax.fori_loop(0, NPT * cout // 16, initv, 0)

        nch = lax.shift_right_arithmetic(cnt + CHS - 1, 9)

        def chunk(ci, _):
            r0 = start + ci * CHS
            pltpu.sync_copy(h_hbm.at[pl.ds(_m8(r0 * cout), CHS * cout)], hbuf)
            pltpu.sync_copy(sd_hbm.at[pl.ds(_m8(r0), CHS)],
                            dbuf.at[pl.ds(0, CHS)])
            rem = cnt - ci * CHS

            def edge(e, _2):
                d = dbuf[pl.ds(e, 16)][0]
                li = jnp.clip(d - lo, 0, NPT - 1)
                base = li * cout
                hb = e * cout
                ok = e < rem
                for kk in range(cout // 16):
                    hv = hbuf[pl.ds(hb + kk * 16, 16)]
                    hv = jnp.where(ok, hv, ninf)
                    av = acc[pl.ds(base + kk * 16, 16)]
                    acc[pl.ds(base + kk * 16, 16)] = jnp.maximum(av, hv)
                return 0

            lax.fori_loop(0, CHS, edge, 0)
            return 0

        lax.fori_loop(0, nch, chunk, 0)
        pltpu.sync_copy(acc, out_hbm.at[pl.ds(_m8(lo * cout), NPT * cout)])

    return k(Hf, sdstg, counts)


def _sc_gather(y, ssrc, sdstg, counts):
    cin = y.shape[1]
    CHG = 128
    yin = jnp.zeros((y.shape[0], 128), jnp.float32).at[:, :cin].set(y)
    xshape = (CHG, 128)

    @functools.partial(
        pl.kernel, mesh=_mesh(),
        compiler_params=pltpu.CompilerParams(needs_layout_passes=False),
        out_type=jax.ShapeDtypeStruct((TOT2 * 2 * cin,), jnp.float32),
        scratch_types=[pltpu.VMEM((NTILE * 16,), jnp.int32),
                       pltpu.VMEM((CHG,), jnp.int32),
                       pltpu.VMEM((CHG,), jnp.int32),
                       pltpu.VMEM(xshape, jnp.float32),
                       pltpu.VMEM(xshape, jnp.float32),
                       pltpu.VMEM((CHG * 2 * cin,), jnp.float32),
                       pltpu.SemaphoreType.DMA,
                       pltpu.SemaphoreType.DMA])
    def k(y_hbm, src_hbm, dst_hbm, cnt_hbm, cat_hbm,
          cvm, ibuf, jbuf, xib, xjb, cbuf, sem1, sem2):
        w = _wid()
        pltpu.sync_copy(cnt_hbm, cvm)
        start = _prefix(cvm, w)
        cnt = cvm[pl.ds(w * 16, 16)][0]
        nch = lax.shift_right_arithmetic(cnt + CHG - 1, 7)
        lanes = lax.iota(jnp.int32, 16)

        def chunk(ci, _):
            r0 = start + ci * CHG
            pltpu.sync_copy(dst_hbm.at[pl.ds(_m8(r0), CHG)], ibuf)
            pltpu.sync_copy(src_hbm.at[pl.ds(_m8(r0), CHG)], jbuf)
            rem = cnt - ci * CHG

            def sanitize(j, _2):
                pos = j * 16 + lanes
                ok = pos < rem
                iv = ibuf[pl.ds(j * 16, 16)]
                jv = jbuf[pl.ds(j * 16, 16)]
                ibuf[pl.ds(j * 16, 16)] = jnp.where(ok, iv, 0)
                jbuf[pl.ds(j * 16, 16)] = jnp.where(ok, jv, 0)
                return 0

            lax.fori_loop(0, CHG // 16, sanitize, 0)
            cp1 = pltpu.async_copy(y_hbm.at[ibuf], xib, sem1)
            cp2 = pltpu.async_copy(y_hbm.at[jbuf], xjb, sem2)
            cp1.wait()
            cp2.wait()

            if cin >= 16:
                def row(r, _3):
                    for kk in range(cin // 16):
                        xi = xib[r, pl.ds(kk * 16, 16)]
                        xj = xjb[r, pl.ds(kk * 16, 16)]
                        cbuf[pl.ds(r * 2 * cin + kk * 16, 16)] = xi
                        cbuf[pl.ds(r * 2 * cin + cin + kk * 16, 16)] = xj - xi
                    return 0

                lax.fori_loop(0, CHG, row, 0)
            elif cin == 8:
                def row8(r, _3):
                    xi = xib[r, pl.ds(0, 16)]
                    xj = xjb[r, pl.ds(0, 16)]
                    base = r * 16
                    plsc.store_scatter(cbuf, [base + lanes], xi)
                    plsc.store_scatter(cbuf, [base + 8 + lanes], xj - xi,
                                       mask=lanes < 8)
                    return 0

                lax.fori_loop(0, CHG, row8, 0)
            else:
                def row1(r, _3):
                    xi = xib[r, pl.ds(0, 16)]
                    xj = xjb[r, pl.ds(0, 16)]
                    base = r * 2
                    plsc.store_scatter(cbuf, [base + lanes], xi,
                                       mask=lanes < 1)
                    plsc.store_scatter(cbuf, [base + 1 + lanes], xj - xi,
                                       mask=lanes < 1)
                    return 0

                lax.fori_loop(0, CHG, row1, 0)

            pltpu.sync_copy(cbuf,
                            cat_hbm.at[pl.ds(_m8(r0 * 2 * cin),
                                             CHG * 2 * cin)])
            return 0

        lax.fori_loop(0, nch, chunk, 0)

    return k(yin, ssrc, sdstg, counts).reshape(TOT2, 2 * cin)


def _edge_mlp_kernel(cat_ref, wa_ref, ba_ref, wb_ref, bb_ref, out_ref):
    h = jnp.maximum(lax.dot(cat_ref[...], wa_ref[...]) + ba_ref[...], 0.0)
    out_ref[...] = lax.dot(h, wb_ref[...]) + bb_ref[...]


def _edge_mlp(CAT, Wa, ba, Wb, bb):
    Ep, k2 = CAT.shape
    cm = Wa.shape[1]
    n = Wb.shape[1]
    assert Ep % BE == 0
    return pl.pallas_call(
        _edge_mlp_kernel,
        grid=(Ep // BE,),
        in_specs=[pl.BlockSpec((BE, k2), lambda i: (i, 0)),
                  pl.BlockSpec((k2, cm), lambda i: (0, 0)),
                  pl.BlockSpec((1, cm), lambda i: (0, 0)),
                  pl.BlockSpec((cm, n), lambda i: (0, 0)),
                  pl.BlockSpec((1, n), lambda i: (0, 0))],
        out_specs=pl.BlockSpec((BE, n), lambda i: (i, 0)),
        out_shape=jax.ShapeDtypeStruct((Ep, n), jnp.float32),
    )(CAT, Wa, ba.reshape(1, -1), Wb, bb.reshape(1, -1))


def _edge_conv_fine(y, ssrc, sdstg, counts, Wa, ba, Wb, bb):
    cout = Wb.shape[1]
    cpad = max(16, cout)
    if cpad != cout:
        Wbp = jnp.zeros((Wb.shape[0], cpad), jnp.float32).at[:, :cout].set(Wb)
        bbp = jnp.zeros((cpad,), jnp.float32).at[:cout].set(bb)
    else:
        Wbp, bbp = Wb, bb
    CAT = _sc_gather(y, ssrc, sdstg, counts)
    H = _edge_mlp(CAT, Wa, ba, Wbp, bbp)
    acc = _sc_scatter(H, sdstg, counts, cpad).reshape(N_NODES, cpad)[:, :cout]
    return jnp.where(jnp.isfinite(acc), acc, 0.0)


def _bn(x, g, b):
    m = jnp.mean(x, axis=0)
    v = jnp.var(x, axis=0)
    return (x - m) * lax.rsqrt(v + 1e-5) * g + b


PD = 232
SP = 256


def _pair_kernel(cat_ref, mask_ref, wa_ref, ba_ref, wb_ref, bb_ref, out_ref):
    h = jnp.maximum(lax.dot(cat_ref[...], wa_ref[...]) + ba_ref[...], 0.0)
    h = lax.dot(h, wb_ref[...]) + bb_ref[...]
    h = jnp.where(mask_ref[...] > 0.5, h, -jnp.inf)
    for i in range(8):
        mx = jnp.max(h[i * SP:(i + 1) * SP], axis=0, keepdims=True)
        out_ref[i:i + 1, :] = jnp.where(jnp.isfinite(mx), mx, 0.0)


def _pair_conv(hin, maskf, Wa, ba, Wb, bb):
    cm = Wa.shape[1]
    n = Wb.shape[1]
    hpad = jnp.zeros((SP, 32), jnp.float32).at[:N_CLUST].set(hin)
    xi = jnp.broadcast_to(hpad[:PD, None, :], (PD, SP, 32))
    xj = jnp.broadcast_to(hpad[None, :, :], (PD, SP, 32))
    cat = jnp.concatenate([xi, xj - xi], axis=2).reshape(PD * SP, 64)
    out = pl.pallas_call(
        _pair_kernel,
        grid=(PD // 8,),
        in_specs=[pl.BlockSpec((8 * SP, 64), lambda i: (i, 0)),
                  pl.BlockSpec((8 * SP, 1), lambda i: (i, 0)),
                  pl.BlockSpec((64, cm), lambda i: (0, 0)),
                  pl.BlockSpec((1, cm), lambda i: (0, 0)),
                  pl.BlockSpec((cm, n), lambda i: (0, 0)),
                  pl.BlockSpec((1, n), lambda i: (0, 0))],
        out_specs=pl.BlockSpec((8, n), lambda i: (i, 0)),
        out_shape=jax.ShapeDtypeStruct((PD, n), jnp.float32),
    )(cat, maskf, Wa, ba.reshape(1, -1), Wb, bb.reshape(1, -1))
    return out[:N_CLUST]


def kernel(x, edge_index, pos, batch, W1a, b1a, W1b, b1b, g1, be1, W2a, b2a,
           W2b, b2b, g2, be2, W3a, b3a, W3b, b3b, g3, be3, W4a, b4a, W4b, b4b,
           g4, be4, W5a, b5a, W5b, b5b, g5, be5, W6a, b6a, W6b, b6b, g6, be6,
           W7a, b7a, W7b, b7b, g7, be7, Wfc):
    src = edge_index[0]
    dst = edge_index[1]
    elu = jax.nn.elu

    counts = _sc_hist(dst)
    ssrc, sdstg = _sc_part(src, dst, counts)

    h = _bn(elu(_edge_conv_fine(x, ssrc, sdstg, counts, W1a, b1a, W1b, b1b)), g1, be1)
    h = _bn(elu(_edge_conv_fine(h, ssrc, sdstg, counts, W2a, b2a, W2b, b2b)), g2, be2)
    sc = h
    h = _bn(elu(_edge_conv_fine(h, ssrc, sdstg, counts, W3a, b3a, W3b, b3b)), g3, be3)
    h = _bn(elu(_edge_conv_fine(h, ssrc, sdstg, counts, W4a, b4a, W4b, b4b)), g4, be4)
    h = h + sc
    h = _bn(elu(_edge_conv_fine(h, ssrc, sdstg, counts, W5a, b5a, W5b, b5b)), g5, be5)

    cx = jnp.floor(pos[:, 0] / 16.0).astype(jnp.int32)
    cy = jnp.floor(pos[:, 1] / 12.0).astype(jnp.int32)
    cluster = cx + 15 * cy
    hp = jax.ops.segment_max(h, cluster, num_segments=N_CLUST)
    hp = jnp.where(jnp.isfinite(hp), hp, 0.0)
    cnt = jax.ops.segment_sum(jnp.ones((N_NODES, 1), dtype=jnp.float32),
                              cluster, num_segments=N_CLUST)
    ppos = jax.ops.segment_sum(pos, cluster,
                               num_segments=N_CLUST) / jnp.maximum(cnt, 1.0)

    pair = cluster[dst] * N_CLUST + cluster[src]
    prescnt = jax.ops.segment_sum(jnp.ones((pair.shape[0],), jnp.float32),
                                  pair, num_segments=N_CLUST * N_CLUST)
    pres = (prescnt > 0.0).reshape(N_CLUST, N_CLUST)
    pres = pres & ~jnp.eye(N_CLUST, dtype=bool)
    maskf = jnp.zeros((PD, SP), jnp.float32).at[:N_CLUST, :N_CLUST].set(
        pres.astype(jnp.float32)).reshape(PD * SP, 1)

    sc = hp
    h = _bn(elu(_pair_conv(hp, maskf, W6a, b6a, W6b, b6b)), g6, be6)
    h = _bn(elu(_pair_conv(h, maskf, W7a, b7a, W7b, b7b)), g7, be7)
    h = h + sc

    gx = jnp.clip(jnp.floor(ppos[:, 0] / 60.0).astype(jnp.int32), 0, 3)
    gy = jnp.clip(jnp.floor(ppos[:, 1] / 45.0).astype(jnp.int32), 0, 3)
    cell = gx + 4 * gy
    occ = cnt[:, 0] > 0
    hm = jnp.where(occ[:, None], h, -jnp.inf)
    out = jax.ops.segment_max(hm, cell, num_segments=16)
    out = jnp.where(jnp.isfinite(out), out, 0.0)
    flat = out.reshape(1, 32 * 16)
    return flat @ Wfc
